# baseline (speedup 1.0000x reference)
import functools
import math

import jax
import jax.numpy as jnp
from jax import lax
from jax.experimental import pallas as pl
from jax.experimental.pallas import tpu as pltpu
from jax.experimental.pallas import tpu_sc as plsc

D_MODEL = 64
SCALE = math.sqrt(D_MODEL)
VOCAB = 1000000
TBLK = 1024
NTBLK = 977
T1_ROWS = TBLK * NTBLK
SEQ = 200
BATCH = 4096


def _transpose_body(tt_ref, t1_ref):
    t = jnp.transpose(tt_ref[...], (1, 0)) * SCALE
    t1_ref[:, 0:D_MODEL] = t
    t1_ref[:, D_MODEL:128] = t


def _transpose_table(tt):
    return pl.pallas_call(
        _transpose_body,
        grid=(NTBLK,),
        in_specs=[pl.BlockSpec((D_MODEL, TBLK), lambda i: (0, i))],
        out_specs=pl.BlockSpec((TBLK, 128), lambda i: (i, 0)),
        out_shape=jax.ShapeDtypeStruct((T1_ROWS, 128), jnp.float32),
    )(tt)


def _build_gather():
    info = plsc.get_sparse_core_info()
    nc, ns = info.num_cores, info.num_subcores
    nw = nc * ns
    n_sblk = BATCH // 128
    ntask = (SEQ // 8) * n_sblk // nw
    mesh = plsc.VectorSubcoreMesh(core_axis_name="c", subcore_axis_name="s")

    @functools.partial(
        pl.kernel,
        mesh=mesh,
        out_type=jax.ShapeDtypeStruct(
            (SEQ, 8, BATCH // 128, 8, 128), jnp.float32
        ),
        scratch_types=[
            pltpu.VMEM((8, 128), jnp.int32),
            pltpu.VMEM((128, 128), jnp.float32),
            pltpu.VMEM((8, 8, 128), jnp.float32),
            pltpu.SemaphoreType.DMA,
        ],
        compiler_params=pltpu.CompilerParams(
            use_tc_tiling_on_sc=False, needs_layout_passes=False
        ),
    )
    def gkern(t1_hbm, xt_hbm, out_hbm, xtile, gdst, slab, sem):
        wid = lax.axis_index("s") * nc + lax.axis_index("c")
        iota = lax.iota(jnp.int32, 16)

        def task_body(k, carry):
            t = wid * ntask + k
            pb = t // n_sblk
            sb = t % n_sblk
            p0 = pb * 8
            s0 = sb * 128
            pltpu.sync_copy(xt_hbm.at[pl.ds(p0, 8), pl.ds(s0, 128)], xtile)

            def p_body(p, carry2):
                pltpu.async_copy(
                    t1_hbm.at[xtile.at[p]], gdst, sem
                ).wait()

                def d_body(d, carry3):
                    dvec = jnp.zeros((16,), jnp.int32) + d
                    di = d // 8
                    dr = d % 8
                    for g in range(8):
                        rowi = iota + g * 16
                        slab[di, dr, pl.ds(g * 16, 16)] = plsc.load_gather(
                            gdst, [rowi, dvec]
                        )
                    return carry3

                lax.fori_loop(0, D_MODEL, d_body, 0, unroll=2)
                pltpu.sync_copy(slab, out_hbm.at[p0 + p, :, sb, :, :])
                return carry2

            lax.fori_loop(0, 8, p_body, 0)
            return carry

        lax.fori_loop(0, ntask, task_body, 0)

    return gkern


def kernel(x, emb_table):
    tt = emb_table.T
    xt = x.T.astype(jnp.int32)
    t1 = _transpose_table(tt)
    out5 = _build_gather()(t1, xt)
    out = jnp.transpose(out5, (2, 4, 0, 1, 3))
    return out.reshape(BATCH, SEQ, D_MODEL)


# speedup vs baseline: 1.4052x; 1.4052x over previous
import functools
import math

import jax
import jax.numpy as jnp
from jax import lax
from jax.experimental import pallas as pl
from jax.experimental.pallas import tpu as pltpu
from jax.experimental.pallas import tpu_sc as plsc

D_MODEL = 64
SCALE = math.sqrt(D_MODEL)
VOCAB = 1000000
TBLK = 8192
NTBLK = 123
T1_ROWS = TBLK * NTBLK
SEQ = 200
BATCH = 4096


def _transpose_body(tt_ref, t1_ref):
    eye = jnp.eye(D_MODEL, dtype=jnp.float32) * SCALE
    t = lax.dot_general(
        tt_ref[...], eye, (((0,), (0,)), ((), ())),
        preferred_element_type=jnp.float32,
    )
    t1_ref[:, 0:D_MODEL] = t
    t1_ref[:, D_MODEL:128] = t


def _transpose_table(tt):
    return pl.pallas_call(
        _transpose_body,
        grid=(NTBLK,),
        in_specs=[pl.BlockSpec((D_MODEL, TBLK), lambda i: (0, i))],
        out_specs=pl.BlockSpec((TBLK, 128), lambda i: (i, 0)),
        out_shape=jax.ShapeDtypeStruct((T1_ROWS, 128), jnp.float32),
    )(tt)


def _build_gather():
    info = plsc.get_sparse_core_info()
    nc, ns = info.num_cores, info.num_subcores
    nw = nc * ns
    n_sblk = BATCH // 128
    ntask = (SEQ // 8) * n_sblk // nw
    mesh = plsc.VectorSubcoreMesh(core_axis_name="c", subcore_axis_name="s")

    @functools.partial(
        pl.kernel,
        mesh=mesh,
        out_type=jax.ShapeDtypeStruct(
            (SEQ, 8, BATCH // 128, 8, 128), jnp.float32
        ),
        scratch_types=[
            pltpu.VMEM((8, 128), jnp.int32),
            pltpu.VMEM((128, 128), jnp.float32),
            pltpu.VMEM((128, 128), jnp.float32),
            pltpu.VMEM((8, 8, 128), jnp.float32),
            pltpu.SemaphoreType.DMA,
        ],
        compiler_params=pltpu.CompilerParams(
            use_tc_tiling_on_sc=False, needs_layout_passes=False
        ),
    )
    def gkern(t1_hbm, xt_hbm, out_hbm, xtile, gdst0, gdst1, slab, gsem):
        wid = lax.axis_index("s") * nc + lax.axis_index("c")
        iota = lax.iota(jnp.int32, 16)

        def fire(p, gdst):
            pltpu.make_async_copy(
                t1_hbm.at[xtile.at[p]], gdst, gsem
            ).start()

        def wait(p, gdst):
            pltpu.make_async_copy(
                t1_hbm.at[xtile.at[p]], gdst, gsem
            ).wait()

        def transpose_and_store(gdst, p0p, sb):
            def d_body(d, carry3):
                dvec = jnp.zeros((16,), jnp.int32) + d
                di = d // 8
                dr = d % 8
                for g in range(8):
                    rowi = iota + g * 16
                    slab[di, dr, pl.ds(g * 16, 16)] = plsc.load_gather(
                        gdst, [rowi, dvec]
                    )
                return carry3

            lax.fori_loop(0, D_MODEL, d_body, 0, unroll=4)
            pltpu.sync_copy(slab, out_hbm.at[p0p, :, sb, :, :])

        def task_body(k, carry):
            t = wid * ntask + k
            pb = t // n_sblk
            sb = t % n_sblk
            p0 = pb * 8
            s0 = sb * 128
            pltpu.sync_copy(xt_hbm.at[pl.ds(p0, 8), pl.ds(s0, 128)], xtile)
            fire(0, gdst0)

            def pair_body(j, carry2):
                p = 2 * j
                wait(p, gdst0)
                fire(p + 1, gdst1)
                transpose_and_store(gdst0, p0 + p, sb)
                wait(p + 1, gdst1)

                @pl.when(j < 3)
                def _():
                    fire(p + 2, gdst0)

                transpose_and_store(gdst1, p0 + p + 1, sb)
                return carry2

            lax.fori_loop(0, 4, pair_body, 0)
            return carry

        lax.fori_loop(0, ntask, task_body, 0)

    return gkern


def kernel(x, emb_table):
    tt = emb_table.T
    xt = x.T.astype(jnp.int32)
    t1 = _transpose_table(tt)
    out5 = _build_gather()(t1, xt)
    out = jnp.transpose(out5, (2, 4, 0, 1, 3))
    return out.reshape(BATCH, SEQ, D_MODEL)


# speedup vs baseline: 1.4529x; 1.0339x over previous
import functools
import math

import jax
import jax.numpy as jnp
from jax import lax
from jax.experimental import pallas as pl
from jax.experimental.pallas import tpu as pltpu
from jax.experimental.pallas import tpu_sc as plsc

D_MODEL = 64
SCALE = math.sqrt(D_MODEL)
VOCAB = 1000000
TBLK = 8192
NTBLK = 123
T1_ROWS = TBLK * NTBLK
SEQ = 200
BATCH = 4096


def _transpose_body(tt_ref, t1_ref):
    eye = jnp.eye(D_MODEL, dtype=jnp.float32) * SCALE
    t = lax.dot_general(
        tt_ref[...], eye, (((0,), (0,)), ((), ())),
        preferred_element_type=jnp.float32,
    )
    t1_ref[:, 0:D_MODEL] = t
    t1_ref[:, D_MODEL:128] = t


def _transpose_table(tt):
    return pl.pallas_call(
        _transpose_body,
        grid=(NTBLK,),
        in_specs=[pl.BlockSpec((D_MODEL, TBLK), lambda i: (0, i))],
        out_specs=pl.BlockSpec((TBLK, 128), lambda i: (i, 0)),
        out_shape=jax.ShapeDtypeStruct((T1_ROWS, 128), jnp.float32),
    )(tt)


def _build_gather():
    info = plsc.get_sparse_core_info()
    nc, ns = info.num_cores, info.num_subcores
    nw = nc * ns
    n_sblk = BATCH // 128
    ntask = (SEQ // 8) * n_sblk // nw
    mesh = plsc.VectorSubcoreMesh(core_axis_name="c", subcore_axis_name="s")

    @functools.partial(
        pl.kernel,
        mesh=mesh,
        out_type=jax.ShapeDtypeStruct(
            (SEQ, 8, BATCH // 128, 8, 128), jnp.float32
        ),
        scratch_types=[
            pltpu.VMEM((8, 128), jnp.int32),
            pltpu.VMEM((128, 128), jnp.float32),
            pltpu.VMEM((128, 128), jnp.float32),
            pltpu.VMEM((8, 8, 128), jnp.float32),
            pltpu.VMEM((8, 8, 128), jnp.float32),
            pltpu.SemaphoreType.DMA,
            pltpu.SemaphoreType.DMA,
        ],
        compiler_params=pltpu.CompilerParams(
            use_tc_tiling_on_sc=False, needs_layout_passes=False
        ),
    )
    def gkern(t1_hbm, xt_hbm, out_hbm, xtile, gdst0, gdst1, slab0, slab1,
              gsem, wsem):
        wid = lax.axis_index("s") * nc + lax.axis_index("c")
        iota = lax.iota(jnp.int32, 16)
        rows = tuple(iota + g * 16 for g in range(8))

        def fire(p, gdst):
            pltpu.make_async_copy(
                t1_hbm.at[xtile.at[p]], gdst, gsem
            ).start()

        def wait(p, gdst):
            pltpu.make_async_copy(
                t1_hbm.at[xtile.at[p]], gdst, gsem
            ).wait()

        def transpose(gdst, slab):
            def d_body(d, rs):
                dvec = jnp.zeros((16,), jnp.int32) + d
                di = d // 8
                dr = d % 8
                for g in range(8):
                    slab[di, dr, pl.ds(g * 16, 16)] = plsc.load_gather(
                        gdst, [rs[g], dvec]
                    )
                return rs

            lax.fori_loop(0, D_MODEL, d_body, rows, unroll=8)

        def start_write(slab, p0p, sb):
            pltpu.make_async_copy(
                slab, out_hbm.at[p0p, :, sb, :, :], wsem
            ).start()

        def wait_write(slab, p0p, sb):
            pltpu.make_async_copy(
                slab, out_hbm.at[p0p, :, sb, :, :], wsem
            ).wait()

        def task_body(k, carry):
            t = wid * ntask + k
            pb = t // n_sblk
            sb = t % n_sblk
            p0 = pb * 8
            s0 = sb * 128
            pltpu.sync_copy(xt_hbm.at[pl.ds(p0, 8), pl.ds(s0, 128)], xtile)
            fire(0, gdst0)

            def pair_body(j, carry2):
                p = 2 * j
                wait(p, gdst0)
                fire(p + 1, gdst1)

                @pl.when(j > 0)
                def _():
                    wait_write(slab0, p0 + p - 2, sb)

                transpose(gdst0, slab0)
                start_write(slab0, p0 + p, sb)
                wait(p + 1, gdst1)

                @pl.when(j < 3)
                def _():
                    fire(p + 2, gdst0)

                @pl.when(j > 0)
                def _():
                    wait_write(slab1, p0 + p - 1, sb)

                transpose(gdst1, slab1)
                start_write(slab1, p0 + p + 1, sb)
                return carry2

            lax.fori_loop(0, 4, pair_body, 0)
            wait_write(slab0, p0 + 6, sb)
            wait_write(slab1, p0 + 7, sb)
            return carry

        lax.fori_loop(0, ntask, task_body, 0)

    return gkern


def kernel(x, emb_table):
    tt = emb_table.T
    xt = x.T.astype(jnp.int32)
    t1 = _transpose_table(tt)
    out5 = _build_gather()(t1, xt)
    out = jnp.transpose(out5, (2, 4, 0, 1, 3))
    return out.reshape(BATCH, SEQ, D_MODEL)


# speedup vs baseline: 2.9124x; 2.0045x over previous
import functools
import math

import jax
import jax.numpy as jnp
from jax import lax
from jax.experimental import pallas as pl
from jax.experimental.pallas import tpu as pltpu
from jax.experimental.pallas import tpu_sc as plsc

D_MODEL = 64
SCALE = math.sqrt(D_MODEL)
VOCAB = 1000000
TBLK = 8192
NTBLK = 123
T1_ROWS = TBLK * NTBLK
SEQ = 200
BATCH = 4096


def _transpose_body(tt_ref, t1_ref):
    eye = jnp.eye(D_MODEL, dtype=jnp.float32) * SCALE
    t = lax.dot_general(
        tt_ref[...], eye, (((0,), (0,)), ((), ())),
        preferred_element_type=jnp.float32,
    )
    t1_ref[:, 0:D_MODEL] = t
    t1_ref[:, D_MODEL:128] = t


def _transpose_table(tt):
    return pl.pallas_call(
        _transpose_body,
        grid=(NTBLK,),
        in_specs=[pl.BlockSpec((D_MODEL, TBLK), lambda i: (0, i))],
        out_specs=pl.BlockSpec((TBLK, 128), lambda i: (i, 0)),
        out_shape=jax.ShapeDtypeStruct((T1_ROWS, 128), jnp.float32),
    )(tt)


def _build_gather():
    info = plsc.get_sparse_core_info()
    nc, ns = info.num_cores, info.num_subcores
    nw = nc * ns
    n_sblk = BATCH // 128
    ntask = (SEQ // 8) * n_sblk // nw
    mesh = plsc.VectorSubcoreMesh(core_axis_name="c", subcore_axis_name="s")

    @functools.partial(
        pl.kernel,
        mesh=mesh,
        out_type=jax.ShapeDtypeStruct(
            (SEQ, 8, BATCH // 128, 8, 128), jnp.float32
        ),
        scratch_types=[
            pltpu.VMEM((8, 128), jnp.int32),
            pltpu.VMEM((128, 128), jnp.float32),
            pltpu.VMEM((128, 128), jnp.float32),
            pltpu.VMEM((8, 8, 128), jnp.float32),
            pltpu.VMEM((8, 8, 128), jnp.float32),
            pltpu.SemaphoreType.DMA,
            pltpu.SemaphoreType.DMA,
        ],
        compiler_params=pltpu.CompilerParams(
            use_tc_tiling_on_sc=False, needs_layout_passes=False
        ),
    )
    def gkern(t1_hbm, xt_hbm, out_hbm, xtile, gdst0, gdst1, slab0, slab1,
              gsem, wsem):
        wid = lax.axis_index("s") * nc + lax.axis_index("c")
        iota = lax.iota(jnp.int32, 16)
        rows = tuple(iota + g * 16 for g in range(8))

        def fire(p, gdst):
            pltpu.make_async_copy(
                t1_hbm.at[xtile.at[p]], gdst, gsem
            ).start()

        def wait(p, gdst):
            pltpu.make_async_copy(
                t1_hbm.at[xtile.at[p]], gdst, gsem
            ).wait()

        def transpose(gdst, slab):
            def d_body(d0, rs):
                cvec = jnp.bitwise_and(iota + d0, 63)
                di = lax.shift_right_logical(cvec, 3)
                dr = jnp.bitwise_and(cvec, 7)
                for g in range(8):
                    vals = plsc.load_gather(gdst, [rs[g], cvec])
                    plsc.store_scatter(slab, [di, dr, rs[g]], vals)
                return rs

            lax.fori_loop(0, D_MODEL, d_body, rows, unroll=8)

        def start_write(slab, p0p, sb):
            pltpu.make_async_copy(
                slab, out_hbm.at[p0p, :, sb, :, :], wsem
            ).start()

        def wait_write(slab, p0p, sb):
            pltpu.make_async_copy(
                slab, out_hbm.at[p0p, :, sb, :, :], wsem
            ).wait()

        def task_body(k, carry):
            t = wid * ntask + k
            pb = t // n_sblk
            sb = t % n_sblk
            p0 = pb * 8
            s0 = sb * 128
            pltpu.sync_copy(xt_hbm.at[pl.ds(p0, 8), pl.ds(s0, 128)], xtile)
            fire(0, gdst0)

            def pair_body(j, carry2):
                p = 2 * j
                wait(p, gdst0)
                fire(p + 1, gdst1)

                @pl.when(j > 0)
                def _():
                    wait_write(slab0, p0 + p - 2, sb)

                transpose(gdst0, slab0)
                start_write(slab0, p0 + p, sb)
                wait(p + 1, gdst1)

                @pl.when(j < 3)
                def _():
                    fire(p + 2, gdst0)

                @pl.when(j > 0)
                def _():
                    wait_write(slab1, p0 + p - 1, sb)

                transpose(gdst1, slab1)
                start_write(slab1, p0 + p + 1, sb)
                return carry2

            lax.fori_loop(0, 4, pair_body, 0)
            wait_write(slab0, p0 + 6, sb)
            wait_write(slab1, p0 + 7, sb)
            return carry

        lax.fori_loop(0, ntask, task_body, 0)

    return gkern


def kernel(x, emb_table):
    tt = emb_table.T
    xt = x.T.astype(jnp.int32)
    t1 = _transpose_table(tt)
    out5 = _build_gather()(t1, xt)
    out = jnp.transpose(out5, (2, 4, 0, 1, 3))
    return out.reshape(BATCH, SEQ, D_MODEL)


# speedup vs baseline: 3.1131x; 1.0689x over previous
import functools
import math

import jax
import jax.numpy as jnp
from jax import lax
from jax.experimental import pallas as pl
from jax.experimental.pallas import tpu as pltpu
from jax.experimental.pallas import tpu_sc as plsc

D_MODEL = 64
SCALE = math.sqrt(D_MODEL)
VOCAB = 1000000
TBLK = 8192
NTBLK = 62
T1_HALF = TBLK * NTBLK
_MAXBLK = VOCAB // TBLK
SEQ = 200
BATCH = 4096


def _transpose_body(lo_ref, hi_ref, t1_ref):
    eye = jnp.eye(D_MODEL, dtype=jnp.float32) * SCALE
    dims = (((0,), (0,)), ((), ()))
    t1_ref[:, 0:D_MODEL] = lax.dot_general(
        lo_ref[...], eye, dims, preferred_element_type=jnp.float32
    )
    t1_ref[:, D_MODEL:128] = lax.dot_general(
        hi_ref[...], eye, dims, preferred_element_type=jnp.float32
    )


def _transpose_table(tt):
    return pl.pallas_call(
        _transpose_body,
        grid=(NTBLK,),
        in_specs=[
            pl.BlockSpec((D_MODEL, TBLK), lambda i: (0, i)),
            pl.BlockSpec(
                (D_MODEL, TBLK),
                lambda i: (0, jnp.minimum(i + NTBLK, _MAXBLK)),
            ),
        ],
        out_specs=pl.BlockSpec((TBLK, 128), lambda i: (i, 0)),
        out_shape=jax.ShapeDtypeStruct((T1_HALF, 128), jnp.float32),
    )(tt, tt)


def _build_gather():
    info = plsc.get_sparse_core_info()
    nc, ns = info.num_cores, info.num_subcores
    nw = nc * ns
    n_sblk = BATCH // 128
    ntask = (SEQ // 8) * n_sblk // nw
    mesh = plsc.VectorSubcoreMesh(core_axis_name="c", subcore_axis_name="s")

    @functools.partial(
        pl.kernel,
        mesh=mesh,
        out_type=jax.ShapeDtypeStruct(
            (SEQ, 8, BATCH // 128, 8, 128), jnp.float32
        ),
        scratch_types=[
            pltpu.VMEM((8, 128), jnp.int32),
            pltpu.VMEM((128,), jnp.int32),
            pltpu.VMEM((128,), jnp.int32),
            pltpu.VMEM((128,), jnp.int32),
            pltpu.VMEM((128,), jnp.int32),
            pltpu.VMEM((128, 128), jnp.float32),
            pltpu.VMEM((128, 128), jnp.float32),
            pltpu.VMEM((8, 8, 128), jnp.float32),
            pltpu.VMEM((8, 8, 128), jnp.float32),
            pltpu.SemaphoreType.DMA,
            pltpu.SemaphoreType.DMA,
        ],
        compiler_params=pltpu.CompilerParams(
            use_tc_tiling_on_sc=False, needs_layout_passes=False
        ),
    )
    def gkern(t1_hbm, xt_hbm, out_hbm, xtile, idx0, idx1, half0, half1,
              gdst0, gdst1, slab0, slab1, gsem, wsem):
        wid = lax.axis_index("s") * nc + lax.axis_index("c")
        iota = lax.iota(jnp.int32, 16)
        rows = tuple(iota + g * 16 for g in range(8))

        def fire(p, idxbuf, halfbuf, gdst):
            for g in range(8):
                v = xtile[p, pl.ds(g * 16, 16)]
                ge = (v >= T1_HALF).astype(jnp.int32)
                idxbuf[pl.ds(g * 16, 16)] = v - ge * T1_HALF
                halfbuf[pl.ds(g * 16, 16)] = lax.shift_left(ge, 6)
            pltpu.make_async_copy(t1_hbm.at[idxbuf], gdst, gsem).start()

        def wait(idxbuf, gdst):
            pltpu.make_async_copy(t1_hbm.at[idxbuf], gdst, gsem).wait()

        def transpose(gdst, slab, halfbuf):
            halves = tuple(halfbuf[pl.ds(g * 16, 16)] for g in range(8))

            def d_body(d0, hs):
                cvec = jnp.bitwise_and(iota + d0, 63)
                di = lax.shift_right_logical(cvec, 3)
                dr = jnp.bitwise_and(cvec, 7)
                for g in range(8):
                    vals = plsc.load_gather(gdst, [rows[g], cvec + hs[g]])
                    plsc.store_scatter(slab, [di, dr, rows[g]], vals)
                return hs

            lax.fori_loop(0, D_MODEL, d_body, halves, unroll=8)

        def start_write(slab, p0p, sb):
            pltpu.make_async_copy(
                slab, out_hbm.at[p0p, :, sb, :, :], wsem
            ).start()

        def wait_write(slab, p0p, sb):
            pltpu.make_async_copy(
                slab, out_hbm.at[p0p, :, sb, :, :], wsem
            ).wait()

        def task_body(k, carry):
            t = wid * ntask + k
            pb = t // n_sblk
            sb = t % n_sblk
            p0 = pb * 8
            s0 = sb * 128
            pltpu.sync_copy(xt_hbm.at[pl.ds(p0, 8), pl.ds(s0, 128)], xtile)
            fire(0, idx0, half0, gdst0)

            def pair_body(j, carry2):
                p = 2 * j
                wait(idx0, gdst0)
                fire(p + 1, idx1, half1, gdst1)

                @pl.when(j > 0)
                def _():
                    wait_write(slab0, p0 + p - 2, sb)

                transpose(gdst0, slab0, half0)
                start_write(slab0, p0 + p, sb)
                wait(idx1, gdst1)

                @pl.when(j < 3)
                def _():
                    fire(p + 2, idx0, half0, gdst0)

                @pl.when(j > 0)
                def _():
                    wait_write(slab1, p0 + p - 1, sb)

                transpose(gdst1, slab1, half1)
                start_write(slab1, p0 + p + 1, sb)
                return carry2

            lax.fori_loop(0, 4, pair_body, 0)
            wait_write(slab0, p0 + 6, sb)
            wait_write(slab1, p0 + 7, sb)
            return carry

        lax.fori_loop(0, ntask, task_body, 0)

    return gkern


def kernel(x, emb_table):
    tt = emb_table.T
    xt = x.T.astype(jnp.int32)
    t1 = _transpose_table(tt)
    out5 = _build_gather()(t1, xt)
    out = jnp.transpose(out5, (2, 4, 0, 1, 3))
    return out.reshape(BATCH, SEQ, D_MODEL)
